# baseline (device time: 14046 ns/iter reference)
import jax
import jax.numpy as jnp
from jax import lax
from jax.experimental import pallas as pl
from jax.experimental.pallas import tpu as pltpu

N_DEV = 8


def kernel(x, w_mat):
    m_per, k = x.shape
    n = w_mat.shape[1]
    n_per = n // N_DEV

    def body(x_hbm, w_hbm, out_hbm, xv, wv, yv, recvv, outv,
             in_sems, out_sems, send_sems, recv_sems):
        me = lax.axis_index("i")

        cx = pltpu.make_async_copy(x_hbm, xv, in_sems.at[N_DEV])
        cx.start()
        w_loads = []
        for d in range(1, N_DEV + 1):
            r = lax.rem(me + d, N_DEV)
            cw = pltpu.make_async_copy(
                w_hbm.at[:, pl.ds(r * n_per, n_per)],
                wv.at[r],
                in_sems.at[r],
            )
            cw.start()
            w_loads.append((r, cw))

        barrier_sem = pltpu.get_barrier_semaphore()
        for d in range(1, N_DEV):
            p = lax.rem(me + d, N_DEV)
            pl.semaphore_signal(
                barrier_sem, inc=1,
                device_id=(p,), device_id_type=pl.DeviceIdType.MESH,
            )
        pl.semaphore_wait(barrier_sem, N_DEV - 1)

        cx.wait()
        xb = xv[...].astype(jnp.bfloat16)

        sends = []
        out_copies = []
        for d, (r, cw) in zip(range(1, N_DEV + 1), w_loads):
            cw.wait()
            block = jnp.dot(xb, wv[r].astype(jnp.bfloat16),
                            preferred_element_type=jnp.float32)
            if d < N_DEV:
                yv[r] = block.astype(jnp.bfloat16)
                c = pltpu.make_async_remote_copy(
                    src_ref=yv.at[r],
                    dst_ref=recvv.at[me],
                    send_sem=send_sems.at[d],
                    recv_sem=recv_sems.at[me],
                    device_id=(r,),
                    device_id_type=pl.DeviceIdType.MESH,
                )
                c.start()
                sends.append(c)
            else:
                outv[pl.ds(me * m_per, m_per), :] = block
                co = pltpu.make_async_copy(
                    outv.at[pl.ds(me * m_per, m_per), :],
                    out_hbm.at[pl.ds(me * m_per, m_per), :],
                    out_sems.at[me],
                )
                co.start()
                out_copies.append(co)

        for d in range(1, N_DEV):
            s = lax.rem(me + d, N_DEV)
            recv = pltpu.make_async_remote_copy(
                src_ref=yv.at[s],
                dst_ref=recvv.at[s],
                send_sem=send_sems.at[d],
                recv_sem=recv_sems.at[s],
                device_id=(s,),
                device_id_type=pl.DeviceIdType.MESH,
            )
            recv.wait_recv()
            outv[pl.ds(s * m_per, m_per), :] = recvv[s].astype(jnp.float32)
            co = pltpu.make_async_copy(
                outv.at[pl.ds(s * m_per, m_per), :],
                out_hbm.at[pl.ds(s * m_per, m_per), :],
                out_sems.at[s],
            )
            co.start()
            out_copies.append(co)

        for co in out_copies:
            co.wait()
        for c in sends:
            c.wait_send()

    return pl.pallas_call(
        body,
        out_shape=jax.ShapeDtypeStruct((N_DEV * m_per, n_per), jnp.float32),
        in_specs=[
            pl.BlockSpec(memory_space=pl.ANY),
            pl.BlockSpec(memory_space=pl.ANY),
        ],
        out_specs=pl.BlockSpec(memory_space=pl.ANY),
        scratch_shapes=[
            pltpu.VMEM((m_per, k), jnp.float32),
            pltpu.VMEM((N_DEV, k, n_per), jnp.float32),
            pltpu.VMEM((N_DEV, m_per, n_per), jnp.bfloat16),
            pltpu.VMEM((N_DEV, m_per, n_per), jnp.bfloat16),
            pltpu.VMEM((N_DEV * m_per, n_per), jnp.float32),
            pltpu.SemaphoreType.DMA((N_DEV + 1,)),
            pltpu.SemaphoreType.DMA((N_DEV,)),
            pltpu.SemaphoreType.DMA((N_DEV,)),
            pltpu.SemaphoreType.DMA((N_DEV,)),
        ],
        compiler_params=pltpu.CompilerParams(collective_id=0),
    )(x, w_mat)


# device time: 9282 ns/iter; 1.5133x vs baseline; 1.5133x over previous
import jax
import jax.numpy as jnp
from jax import lax
from jax.experimental import pallas as pl
from jax.experimental.pallas import tpu as pltpu

N_DEV = 8


def kernel(x, w_mat):
    m_per, k = x.shape
    n = w_mat.shape[1]
    n_per = n // N_DEV

    def body(x_hbm, w_hbm, out_ref, xv, wv, y_ref, recvv,
             in_sems, send_sems, recv_sems):
        me = lax.axis_index("i")

        cx = pltpu.make_async_copy(x_hbm, xv, in_sems.at[N_DEV])
        cx.start()
        w_loads = []
        for d in range(1, N_DEV + 1):
            r = lax.rem(me + d, N_DEV)
            cw = pltpu.make_async_copy(
                w_hbm.at[:, pl.ds(r * n_per, n_per)],
                wv.at[r],
                in_sems.at[r],
            )
            cw.start()
            w_loads.append((r, cw))

        barrier_sem = pltpu.get_barrier_semaphore()
        for d in range(1, N_DEV):
            p = lax.rem(me + d, N_DEV)
            pl.semaphore_signal(
                barrier_sem, inc=1,
                device_id=(p,), device_id_type=pl.DeviceIdType.MESH,
            )

        cx.wait()
        xb = xv[...].astype(jnp.bfloat16)

        sends = []
        for d, (r, cw) in zip(range(1, N_DEV + 1), w_loads):
            cw.wait()
            block = jnp.dot(xb, wv[r].astype(jnp.bfloat16),
                            preferred_element_type=jnp.float32)
            if d < N_DEV:
                y_ref[d] = block.astype(jnp.bfloat16)
                if d == 1:
                    pl.semaphore_wait(barrier_sem, N_DEV - 1)
                c = pltpu.make_async_remote_copy(
                    src_ref=y_ref.at[d],
                    dst_ref=recvv.at[me],
                    send_sem=send_sems.at[d],
                    recv_sem=recv_sems.at[me],
                    device_id=(r,),
                    device_id_type=pl.DeviceIdType.MESH,
                )
                c.start()
                sends.append(c)
            else:
                out_ref[pl.ds(me * m_per, m_per), :] = block

        for d in range(1, N_DEV):
            s = lax.rem(me + N_DEV - d, N_DEV)
            recv = pltpu.make_async_remote_copy(
                src_ref=y_ref.at[min(d, N_DEV - 1)],
                dst_ref=recvv.at[s],
                send_sem=send_sems.at[d],
                recv_sem=recv_sems.at[s],
                device_id=(s,),
                device_id_type=pl.DeviceIdType.MESH,
            )
            recv.wait_recv()
            out_ref[pl.ds(s * m_per, m_per), :] = recvv[s].astype(jnp.float32)

        for c in sends:
            c.wait_send()

    x = pltpu.with_memory_space_constraint(x, pltpu.MemorySpace.HBM)
    w_mat = pltpu.with_memory_space_constraint(w_mat, pltpu.MemorySpace.HBM)

    return pl.pallas_call(
        body,
        out_shape=jax.ShapeDtypeStruct((N_DEV * m_per, n_per), jnp.float32),
        in_specs=[
            pl.BlockSpec(memory_space=pltpu.MemorySpace.HBM),
            pl.BlockSpec(memory_space=pltpu.MemorySpace.HBM),
        ],
        out_specs=pl.BlockSpec(memory_space=pltpu.VMEM),
        scratch_shapes=[
            pltpu.VMEM((m_per, k), jnp.float32),
            pltpu.VMEM((N_DEV, k, n_per), jnp.float32),
            pltpu.VMEM((N_DEV, m_per, n_per), jnp.bfloat16),
            pltpu.VMEM((N_DEV, m_per, n_per), jnp.bfloat16),
            pltpu.SemaphoreType.DMA((N_DEV + 1,)),
            pltpu.SemaphoreType.DMA((N_DEV,)),
            pltpu.SemaphoreType.DMA((N_DEV,)),
        ],
        compiler_params=pltpu.CompilerParams(collective_id=0),
    )(x, w_mat)
